# baseline (speedup 1.0000x reference)
import functools

import jax
import jax.numpy as jnp
from jax.experimental import pallas as pl
from jax.experimental.pallas import tpu as pltpu

_EPS = 1e-3
_KEEP_RATIO = 0.7
_LB = 2048


def _dot1(a, b):
    return jnp.dot(a.astype(jnp.bfloat16), b.astype(jnp.bfloat16),
                   preferred_element_type=jnp.float32)


def _dgT1(a, b):
    return jax.lax.dot_general(
        a.astype(jnp.bfloat16), b.astype(jnp.bfloat16),
        dimension_numbers=(((1,), (1,)), ((), ())),
        preferred_element_type=jnp.float32)


def _main_kernel(gw1_ref, gb1_ref, gw2_ref, gb2_ref,
                 w1_ref, b1_ref, w2_ref, b2_ref,
                 x_ref, c_ref, imp_ref, y_ref,
                 scr, *, inv_l, n_c, nl):
    p = pl.program_id(1)
    l = pl.program_id(2)
    x = x_ref[0]

    @pl.when(p == 0)
    def _():
        lb = x.shape[1]
        hl = lb // 2
        ps = None
        for s in range(2):
            cs = c_ref[0][:, s * hl:(s + 1) * hl]
            xs = x[:, s * hl:(s + 1) * hl]
            h = _dot1(gw1_ref[...], cs) + gb1_ref[...]
            h = jnp.maximum(h, 0.0)
            g = _dot1(gw2_ref[...], h) + gb2_ref[...]
            g = jax.nn.sigmoid(g)
            pss = jnp.concatenate(
                [jnp.sum(xs, axis=1, keepdims=True),
                 jnp.sum(xs * xs, axis=1, keepdims=True),
                 jnp.sum(g, axis=1, keepdims=True),
                 jnp.sum(cs, axis=1, keepdims=True)], axis=1)
            ps = pss if ps is None else ps + pss

        @pl.when(l == 0)
        def _():
            scr[:, 0:4] = ps

        @pl.when(l != 0)
        def _():
            scr[:, 0:4] += ps

    @pl.when((p == 1) & (l == 0))
    def _():
        mu_c = scr[:, 0:1] * inv_l
        ex2 = scr[:, 1:2] * inv_l
        sigma_c = jnp.sqrt(ex2 - mu_c * mu_c + _EPS)
        mu_l = jnp.mean(mu_c, axis=0, keepdims=True)
        ex2_l = jnp.mean(ex2, axis=0, keepdims=True)
        sigma_l = jnp.sqrt(ex2_l - mu_l * mu_l + _EPS)
        gm = scr[:, 2:3] * inv_l
        mu = gm * mu_c + (1.0 - gm) * mu_l
        sigma = gm * sigma_c + (1.0 - gm) * sigma_l
        cond_t = jnp.transpose(scr[:, 3:4] * inv_l)
        h2 = _dgT1(cond_t, w1_ref[...]) + b1_ref[...]
        h2 = jnp.maximum(h2, 0.0)
        gb = _dgT1(h2, w2_ref[...]) + b2_ref[...]
        gbt = jnp.transpose(gb)
        gamma = gbt[:n_c]
        beta = gbt[n_c:]
        a = (1.0 + gamma) / sigma
        scr[:, 4:5] = a
        scr[:, 5:6] = beta - a * mu

    @pl.when(p == 1)
    def _():
        y = scr[:, 4:5] * x + scr[:, 5:6]
        y_ref[0] = y
        pimp = jnp.sum(jnp.abs(y), axis=1, keepdims=True)

        @pl.when(l == 0)
        def _():
            scr[:, 6:7] = pimp

        @pl.when(l != 0)
        def _():
            scr[:, 6:7] += pimp

        @pl.when(l == nl - 1)
        def _():
            imp_ref[0] = jnp.transpose(scr[:, 6:7])


def _mask_zero_kernel(impl_ref, y_hbm, out_hbm,
                      idx_vmem, idx_smem, zbuf, idx_sem, zsem, *, k, n_drop):
    del y_hbm
    b = pl.program_id(0)
    n_c = impl_ref.shape[2]

    @pl.when(b == 0)
    def _():
        zbuf[...] = jnp.zeros_like(zbuf)

    imp_l = impl_ref[0]
    imp_c = jnp.transpose(imp_l)
    vi = jnp.broadcast_to(imp_c, (n_c, n_c))
    vj = jnp.broadcast_to(imp_l, (n_c, n_c))
    ii = jax.lax.broadcasted_iota(jnp.int32, (n_c, n_c), 0)
    jj = jax.lax.broadcasted_iota(jnp.int32, (n_c, n_c), 1)
    jbeats = jnp.where(vj > vi, 1.0,
                       jnp.where((vj == vi) & (jj < ii), 1.0, 0.0))
    rank_col = jnp.sum(jbeats, axis=1, keepdims=True)
    dropped_col = rank_col >= float(k)
    ibeats = jnp.where(vi > vj, 1.0,
                       jnp.where((vi == vj) & (ii < jj), 1.0, 0.0))
    rank_lane = jnp.sum(ibeats, axis=0, keepdims=True)
    dropped_lane = rank_lane >= float(k)
    dr_col = jnp.sum(jnp.where(dropped_lane & (jj < ii), 1.0, 0.0),
                     axis=1, keepdims=True)
    slot = jnp.where((dr_col == jj.astype(jnp.float32)) & dropped_col,
                     1.0, 0.0)
    idx_f = jnp.sum(slot * ii.astype(jnp.float32), axis=0, keepdims=True)
    idx_vmem[...] = idx_f.astype(jnp.int32)
    cp = pltpu.make_async_copy(idx_vmem, idx_smem, idx_sem)
    cp.start()
    cp.wait()
    for i in range(n_drop):
        ch = idx_smem[0, i]
        pltpu.make_async_copy(zbuf, out_hbm.at[b, ch], zsem).start()
    for i in range(n_drop):
        ch = idx_smem[0, i]
        pltpu.make_async_copy(zbuf, out_hbm.at[b, ch], zsem).wait()


def kernel(x16, c16, gate_w1, gate_b1, gate_w2, gate_b2,
           mlp_w1, mlp_b1, mlp_w2, mlp_b2):
    x = x16.astype(jnp.float32)
    c = c16.astype(jnp.float32)
    B, C, L = x.shape
    H = gate_w1.shape[0]
    lb = min(_LB, L)
    nl = L // lb
    k = max(1, int(C * _KEEP_RATIO))
    f32 = jnp.float32

    gw1 = gate_w1.astype(jnp.bfloat16)
    gw2 = gate_w2.astype(jnp.bfloat16)
    w1 = mlp_w1.astype(jnp.bfloat16)
    w2 = mlp_w2.astype(jnp.bfloat16)
    gb1 = gate_b1.reshape(H, 1)
    gb2 = gate_b2.reshape(C, 1)
    b1 = mlp_b1.reshape(1, H)
    b2 = mlp_b2.reshape(1, 2 * C)

    wspec = lambda shp: pl.BlockSpec(shp, lambda b, p, l: (0,) * len(shp))
    imp, y = pl.pallas_call(
        functools.partial(_main_kernel, inv_l=1.0 / L, n_c=C, nl=nl),
        grid=(B, 2, nl),
        in_specs=[
            wspec((H, C)), wspec((H, 1)),
            wspec((C, H)), wspec((C, 1)),
            wspec((H, C)), wspec((1, H)),
            wspec((2 * C, H)), wspec((1, 2 * C)),
            pl.BlockSpec((1, C, lb), lambda b, p, l: (b, 0, l)),
            pl.BlockSpec((1, C, lb),
                         lambda b, p, l: (b, 0, jnp.where(p == 0, l, nl - 1))),
        ],
        out_specs=[
            pl.BlockSpec((1, 1, C), lambda b, p, l: (b, 0, 0)),
            pl.BlockSpec((1, C, lb),
                         lambda b, p, l: (b, 0, jnp.where(p == 0, 0, l))),
        ],
        out_shape=[
            jax.ShapeDtypeStruct((B, 1, C), f32),
            jax.ShapeDtypeStruct((B, C, L), f32),
        ],
        scratch_shapes=[pltpu.VMEM((C, 8), f32)],
        compiler_params=pltpu.CompilerParams(
            dimension_semantics=("parallel", "arbitrary", "arbitrary"),
            vmem_limit_bytes=58 * 1024 * 1024,
        ),
        name="duan_main",
    )(gw1, gb1, gw2, gb2, w1, b1, w2, b2, x, c)

    y_out = pl.pallas_call(
        functools.partial(_mask_zero_kernel, k=k, n_drop=C - k),
        grid=(B,),
        in_specs=[
            pl.BlockSpec((1, 1, C), lambda b: (b, 0, 0)),
            pl.BlockSpec(memory_space=pl.ANY),
        ],
        out_specs=pl.BlockSpec(memory_space=pl.ANY),
        out_shape=jax.ShapeDtypeStruct((B, C, L), f32),
        input_output_aliases={1: 0},
        scratch_shapes=[
            pltpu.VMEM((1, C), jnp.int32),
            pltpu.SMEM((1, C), jnp.int32),
            pltpu.VMEM((L,), f32),
            pltpu.SemaphoreType.DMA,
            pltpu.SemaphoreType.DMA,
        ],
        compiler_params=pltpu.CompilerParams(
            dimension_semantics=("arbitrary",)),
        name="duan_mask_zero",
    )(imp, y)
    return y_out.astype(x16.dtype)


# speedup vs baseline: 1.0999x; 1.0999x over previous
import functools

import jax
import jax.numpy as jnp
from jax.experimental import pallas as pl
from jax.experimental.pallas import tpu as pltpu

_EPS = 1e-3
_KEEP_RATIO = 0.7
_LB1 = 4096
_LB2 = 4096


def _dot1(a, b):
    return jnp.dot(a.astype(jnp.bfloat16), b.astype(jnp.bfloat16),
                   preferred_element_type=jnp.float32)


def _dgT1(a, b):
    return jax.lax.dot_general(
        a.astype(jnp.bfloat16), b.astype(jnp.bfloat16),
        dimension_numbers=(((1,), (1,)), ((), ())),
        preferred_element_type=jnp.float32)


def _stats_kernel(gw1_ref, gb1_ref, gw2_ref, gb2_ref,
                  x_ref, c_ref, sx_ref, sx2_ref, sg_ref, sc_ref):
    l = pl.program_id(1)
    x = x_ref[0]
    c = c_ref[0]
    h = _dot1(gw1_ref[...], c) + gb1_ref[...]
    h = jnp.maximum(h, 0.0)
    g = _dot1(gw2_ref[...], h) + gb2_ref[...]
    g = jax.nn.sigmoid(g)
    psx = jnp.sum(x, axis=1, keepdims=True)
    psx2 = jnp.sum(x * x, axis=1, keepdims=True)
    psg = jnp.sum(g, axis=1, keepdims=True)
    psc = jnp.sum(c, axis=1, keepdims=True)

    @pl.when(l == 0)
    def _():
        sx_ref[0] = psx
        sx2_ref[0] = psx2
        sg_ref[0] = psg
        sc_ref[0] = psc

    @pl.when(l != 0)
    def _():
        sx_ref[0] += psx
        sx2_ref[0] += psx2
        sg_ref[0] += psg
        sc_ref[0] += psc


def _scalar_kernel(sx_ref, sx2_ref, sg_ref, sc_ref,
                   w1_ref, b1_ref, w2_ref, b2_ref,
                   a_ref, b_out_ref, *, inv_l, n_c):
    mu_c = sx_ref[...] * inv_l
    ex2 = sx2_ref[...] * inv_l
    sigma_c = jnp.sqrt(ex2 - mu_c * mu_c + _EPS)
    mu_l = jnp.mean(mu_c, axis=1, keepdims=True)
    ex2_l = jnp.mean(ex2, axis=1, keepdims=True)
    sigma_l = jnp.sqrt(ex2_l - mu_l * mu_l + _EPS)
    gm = sg_ref[...] * inv_l
    mu = gm * mu_c + (1.0 - gm) * mu_l
    sigma = gm * sigma_c + (1.0 - gm) * sigma_l
    cond = sc_ref[...] * inv_l
    h2 = _dgT1(cond, w1_ref[...]) + b1_ref[...]
    h2 = jnp.maximum(h2, 0.0)
    gb = _dgT1(h2, w2_ref[...]) + b2_ref[...]
    gamma = gb[:, :n_c]
    beta = gb[:, n_c:]
    a = (1.0 + gamma) / sigma
    a_ref[...] = a
    b_out_ref[...] = beta - a * mu


def _imp_write_kernel(x_ref, a_ref, b_ref, imp_ref, y_ref):
    l = pl.program_id(1)
    y = a_ref[0] * x_ref[0] + b_ref[0]
    y_ref[0] = y
    p = jnp.sum(jnp.abs(y), axis=1, keepdims=True)

    @pl.when(l == 0)
    def _():
        imp_ref[0] = p

    @pl.when(l != 0)
    def _():
        imp_ref[0] += p


def _mask_zero_kernel(impc_ref, impl_ref, y_hbm, out_hbm,
                      idx_vmem, idx_smem, zbuf, idx_sem, zsem, *, k, n_drop):
    del y_hbm
    b = pl.program_id(0)
    n_c = impc_ref.shape[1]

    @pl.when(b == 0)
    def _():
        zbuf[...] = jnp.zeros_like(zbuf)

    vi = jnp.broadcast_to(impc_ref[0], (n_c, n_c))
    vj = jnp.broadcast_to(impl_ref[0], (n_c, n_c))
    ii = jax.lax.broadcasted_iota(jnp.int32, (n_c, n_c), 0)
    jj = jax.lax.broadcasted_iota(jnp.int32, (n_c, n_c), 1)
    jbeats = jnp.where(vj > vi, 1.0,
                       jnp.where((vj == vi) & (jj < ii), 1.0, 0.0))
    rank_col = jnp.sum(jbeats, axis=1, keepdims=True)
    dropped_col = rank_col >= float(k)
    ibeats = jnp.where(vi > vj, 1.0,
                       jnp.where((vi == vj) & (ii < jj), 1.0, 0.0))
    rank_lane = jnp.sum(ibeats, axis=0, keepdims=True)
    dropped_lane = rank_lane >= float(k)
    dr_col = jnp.sum(jnp.where(dropped_lane & (jj < ii), 1.0, 0.0),
                     axis=1, keepdims=True)
    slot = jnp.where((dr_col == jj.astype(jnp.float32)) & dropped_col,
                     1.0, 0.0)
    idx_f = jnp.sum(slot * ii.astype(jnp.float32), axis=0, keepdims=True)
    idx_vmem[...] = idx_f.astype(jnp.int32)
    cp = pltpu.make_async_copy(idx_vmem, idx_smem, idx_sem)
    cp.start()
    cp.wait()
    for i in range(n_drop):
        ch = idx_smem[0, i]
        pltpu.make_async_copy(zbuf, out_hbm.at[b, ch], zsem).start()
    for i in range(n_drop):
        ch = idx_smem[0, i]
        pltpu.make_async_copy(zbuf, out_hbm.at[b, ch], zsem).wait()


def kernel(x16, c16, gate_w1, gate_b1, gate_w2, gate_b2,
           mlp_w1, mlp_b1, mlp_w2, mlp_b2):
    x = x16.astype(jnp.float32)
    c = c16.astype(jnp.float32)
    B, C, L = x.shape
    H = gate_w1.shape[0]
    lb1 = min(_LB1, L)
    lb2 = min(_LB2, L)
    nl1 = L // lb1
    nl2 = L // lb2
    k = max(1, int(C * _KEEP_RATIO))
    f32 = jnp.float32

    gw1 = gate_w1.astype(jnp.bfloat16)
    gw2 = gate_w2.astype(jnp.bfloat16)
    w1 = mlp_w1.astype(jnp.bfloat16)
    w2 = mlp_w2.astype(jnp.bfloat16)
    gb1 = gate_b1.reshape(H, 1)
    gb2 = gate_b2.reshape(C, 1)
    b1 = mlp_b1.reshape(1, H)
    b2 = mlp_b2.reshape(1, 2 * C)

    wspec = lambda shp: pl.BlockSpec(shp, lambda b, l: (0,) * len(shp))
    sums = pl.pallas_call(
        _stats_kernel,
        grid=(B, nl1),
        in_specs=[
            wspec((H, C)), wspec((H, 1)),
            wspec((C, H)), wspec((C, 1)),
            pl.BlockSpec((1, C, lb1), lambda b, l: (b, 0, l)),
            pl.BlockSpec((1, C, lb1), lambda b, l: (b, 0, l)),
        ],
        out_specs=[pl.BlockSpec((1, C, 1), lambda b, l: (b, 0, 0))] * 4,
        out_shape=[jax.ShapeDtypeStruct((B, C, 1), f32)] * 4,
        compiler_params=pltpu.CompilerParams(
            dimension_semantics=("parallel", "arbitrary"),
            vmem_limit_bytes=50 * 1024 * 1024,
        ),
        name="duan_stats",
    )(gw1, gb1, gw2, gb2, x, c)
    sx, sx2, sg, sc = (s.reshape(B, C) for s in sums)

    w0 = lambda shp: pl.BlockSpec(shp, lambda i: (0,) * len(shp))
    av, bv = pl.pallas_call(
        functools.partial(_scalar_kernel, inv_l=1.0 / L, n_c=C),
        grid=(1,),
        in_specs=[
            w0((B, C)), w0((B, C)), w0((B, C)), w0((B, C)),
            w0((H, C)), w0((1, H)),
            w0((2 * C, H)), w0((1, 2 * C)),
        ],
        out_specs=[w0((B, C)), w0((B, C))],
        out_shape=[jax.ShapeDtypeStruct((B, C), f32)] * 2,
        compiler_params=pltpu.CompilerParams(
            dimension_semantics=("arbitrary",)),
        name="duan_scalars",
    )(sx, sx2, sg, sc, w1, b1, w2, b2)
    av3 = av.reshape(B, C, 1)
    bv3 = bv.reshape(B, C, 1)

    imp, y = pl.pallas_call(
        _imp_write_kernel,
        grid=(B, nl2),
        in_specs=[
            pl.BlockSpec((1, C, lb2), lambda b, l: (b, 0, l)),
            pl.BlockSpec((1, C, 1), lambda b, l: (b, 0, 0)),
            pl.BlockSpec((1, C, 1), lambda b, l: (b, 0, 0)),
        ],
        out_specs=[
            pl.BlockSpec((1, C, 1), lambda b, l: (b, 0, 0)),
            pl.BlockSpec((1, C, lb2), lambda b, l: (b, 0, l)),
        ],
        out_shape=[
            jax.ShapeDtypeStruct((B, C, 1), f32),
            jax.ShapeDtypeStruct((B, C, L), f32),
        ],
        compiler_params=pltpu.CompilerParams(
            dimension_semantics=("parallel", "arbitrary"),
            vmem_limit_bytes=50 * 1024 * 1024,
        ),
        name="duan_imp",
    )(x, av3, bv3)
    impl_t = jnp.transpose(imp, (0, 2, 1))

    y_out = pl.pallas_call(
        functools.partial(_mask_zero_kernel, k=k, n_drop=C - k),
        grid=(B,),
        in_specs=[
            pl.BlockSpec((1, C, 1), lambda b: (b, 0, 0)),
            pl.BlockSpec((1, 1, C), lambda b: (b, 0, 0)),
            pl.BlockSpec(memory_space=pl.ANY),
        ],
        out_specs=pl.BlockSpec(memory_space=pl.ANY),
        out_shape=jax.ShapeDtypeStruct((B, C, L), f32),
        input_output_aliases={2: 0},
        scratch_shapes=[
            pltpu.VMEM((1, C), jnp.int32),
            pltpu.SMEM((1, C), jnp.int32),
            pltpu.VMEM((L,), f32),
            pltpu.SemaphoreType.DMA,
            pltpu.SemaphoreType.DMA,
        ],
        compiler_params=pltpu.CompilerParams(
            dimension_semantics=("arbitrary",)),
        name="duan_mask_zero",
    )(imp, impl_t, y)
    return y_out.astype(x16.dtype)


# speedup vs baseline: 1.1356x; 1.0324x over previous
import functools

import jax
import jax.numpy as jnp
from jax.experimental import pallas as pl
from jax.experimental.pallas import tpu as pltpu

_EPS = 1e-3
_KEEP_RATIO = 0.7
_LB1 = 4096
_LB2 = 4096


def _dot1(a, b):
    return jnp.dot(a.astype(jnp.bfloat16), b.astype(jnp.bfloat16),
                   preferred_element_type=jnp.float32)


def _dgT1(a, b):
    return jax.lax.dot_general(
        a.astype(jnp.bfloat16), b.astype(jnp.bfloat16),
        dimension_numbers=(((1,), (1,)), ((), ())),
        preferred_element_type=jnp.float32)


def _stats_kernel(gw1_ref, gb1_ref, gw2_ref, gb2_ref,
                  x_ref, c_ref, sx_ref, sx2_ref, sg_ref, sc_ref):
    l = pl.program_id(1)
    x = x_ref[0]
    c = c_ref[0]
    h = _dot1(gw1_ref[...], c) + gb1_ref[...]
    h = jnp.maximum(h, 0.0)
    g = _dot1(gw2_ref[...], h) + gb2_ref[...]
    g = jax.nn.sigmoid(g)
    psx = jnp.sum(x, axis=1, keepdims=True)
    psx2 = jnp.sum(x * x, axis=1, keepdims=True)
    psg = jnp.sum(g, axis=1, keepdims=True)
    psc = jnp.sum(c, axis=1, keepdims=True)

    @pl.when(l == 0)
    def _():
        sx_ref[0] = psx
        sx2_ref[0] = psx2
        sg_ref[0] = psg
        sc_ref[0] = psc

    @pl.when(l != 0)
    def _():
        sx_ref[0] += psx
        sx2_ref[0] += psx2
        sg_ref[0] += psg
        sc_ref[0] += psc


def _scalar_kernel(sx_ref, sx2_ref, sg_ref, sc_ref,
                   w1_ref, b1_ref, w2_ref, b2_ref,
                   a_ref, b_out_ref, *, inv_l, n_c):
    mu_c = sx_ref[...] * inv_l
    ex2 = sx2_ref[...] * inv_l
    sigma_c = jnp.sqrt(ex2 - mu_c * mu_c + _EPS)
    mu_l = jnp.mean(mu_c, axis=1, keepdims=True)
    ex2_l = jnp.mean(ex2, axis=1, keepdims=True)
    sigma_l = jnp.sqrt(ex2_l - mu_l * mu_l + _EPS)
    gm = sg_ref[...] * inv_l
    mu = gm * mu_c + (1.0 - gm) * mu_l
    sigma = gm * sigma_c + (1.0 - gm) * sigma_l
    cond = sc_ref[...] * inv_l
    h2 = _dgT1(cond, w1_ref[...]) + b1_ref[...]
    h2 = jnp.maximum(h2, 0.0)
    gb = _dgT1(h2, w2_ref[...]) + b2_ref[...]
    gamma = gb[:, :n_c]
    beta = gb[:, n_c:]
    a = (1.0 + gamma) / sigma
    a_ref[...] = a
    b_out_ref[...] = beta - a * mu


def _imp_write_kernel(x_ref, a_ref, b_ref, imp_ref, y_ref):
    l = pl.program_id(1)
    y = a_ref[0] * x_ref[0] + b_ref[0]
    y_ref[0] = y
    p = jnp.sum(jnp.abs(y), axis=1, keepdims=True)

    @pl.when(l == 0)
    def _():
        imp_ref[0] = p

    @pl.when(l != 0)
    def _():
        imp_ref[0] += p


def _mask_zero_kernel(impc_ref, impl_ref, y_hbm, out_hbm,
                      idx_vmem, idx_smem, zbuf, idx_sem, zsem,
                      *, k, n_drop, n_b):
    del y_hbm
    b = pl.program_id(0)
    n_c = impc_ref.shape[1]

    @pl.when(b == 0)
    def _():
        zbuf[...] = jnp.zeros_like(zbuf)

    vi = jnp.broadcast_to(impc_ref[0], (n_c, n_c))
    vj = jnp.broadcast_to(impl_ref[0], (n_c, n_c))
    ii = jax.lax.broadcasted_iota(jnp.int32, (n_c, n_c), 0)
    jj = jax.lax.broadcasted_iota(jnp.int32, (n_c, n_c), 1)
    jbeats = jnp.where(vj > vi, 1.0,
                       jnp.where((vj == vi) & (jj < ii), 1.0, 0.0))
    rank_col = jnp.sum(jbeats, axis=1, keepdims=True)
    dropped_col = rank_col >= float(k)
    ibeats = jnp.where(vi > vj, 1.0,
                       jnp.where((vi == vj) & (ii < jj), 1.0, 0.0))
    rank_lane = jnp.sum(ibeats, axis=0, keepdims=True)
    dropped_lane = rank_lane >= float(k)
    dr_col = jnp.sum(jnp.where(dropped_lane & (jj < ii), 1.0, 0.0),
                     axis=1, keepdims=True)
    slot = jnp.where((dr_col == jj.astype(jnp.float32)) & dropped_col,
                     1.0, 0.0)
    idx_f = jnp.sum(slot * ii.astype(jnp.float32), axis=0, keepdims=True)
    idx_vmem[...] = idx_f.astype(jnp.int32)
    cp = pltpu.make_async_copy(idx_vmem, idx_smem, idx_sem)
    cp.start()
    cp.wait()
    for i in range(n_drop):
        ch = idx_smem[0, i]
        pltpu.make_async_copy(zbuf, out_hbm.at[b, ch], zsem).start()

    @pl.when(b == n_b - 1)
    def _():
        ch0 = idx_smem[0, 0]
        for i in range(n_b * n_drop):
            pltpu.make_async_copy(zbuf, out_hbm.at[b, ch0], zsem).wait()


def kernel(x16, c16, gate_w1, gate_b1, gate_w2, gate_b2,
           mlp_w1, mlp_b1, mlp_w2, mlp_b2):
    x = x16.astype(jnp.float32)
    c = c16.astype(jnp.float32)
    B, C, L = x.shape
    H = gate_w1.shape[0]
    lb1 = min(_LB1, L)
    lb2 = min(_LB2, L)
    nl1 = L // lb1
    nl2 = L // lb2
    k = max(1, int(C * _KEEP_RATIO))
    f32 = jnp.float32

    gw1 = gate_w1.astype(jnp.bfloat16)
    gw2 = gate_w2.astype(jnp.bfloat16)
    w1 = mlp_w1.astype(jnp.bfloat16)
    w2 = mlp_w2.astype(jnp.bfloat16)
    gb1 = gate_b1.reshape(H, 1)
    gb2 = gate_b2.reshape(C, 1)
    b1 = mlp_b1.reshape(1, H)
    b2 = mlp_b2.reshape(1, 2 * C)

    wspec = lambda shp: pl.BlockSpec(shp, lambda b, l: (0,) * len(shp))
    sums = pl.pallas_call(
        _stats_kernel,
        grid=(B, nl1),
        in_specs=[
            wspec((H, C)), wspec((H, 1)),
            wspec((C, H)), wspec((C, 1)),
            pl.BlockSpec((1, C, lb1), lambda b, l: (b, 0, l)),
            pl.BlockSpec((1, C, lb1), lambda b, l: (b, 0, l)),
        ],
        out_specs=[pl.BlockSpec((1, C, 1), lambda b, l: (b, 0, 0))] * 4,
        out_shape=[jax.ShapeDtypeStruct((B, C, 1), f32)] * 4,
        compiler_params=pltpu.CompilerParams(
            dimension_semantics=("parallel", "arbitrary"),
            vmem_limit_bytes=50 * 1024 * 1024,
        ),
        name="duan_stats",
    )(gw1, gb1, gw2, gb2, x, c)
    sx, sx2, sg, sc = (s.reshape(B, C) for s in sums)

    w0 = lambda shp: pl.BlockSpec(shp, lambda i: (0,) * len(shp))
    av, bv = pl.pallas_call(
        functools.partial(_scalar_kernel, inv_l=1.0 / L, n_c=C),
        grid=(1,),
        in_specs=[
            w0((B, C)), w0((B, C)), w0((B, C)), w0((B, C)),
            w0((H, C)), w0((1, H)),
            w0((2 * C, H)), w0((1, 2 * C)),
        ],
        out_specs=[w0((B, C)), w0((B, C))],
        out_shape=[jax.ShapeDtypeStruct((B, C), f32)] * 2,
        compiler_params=pltpu.CompilerParams(
            dimension_semantics=("arbitrary",)),
        name="duan_scalars",
    )(sx, sx2, sg, sc, w1, b1, w2, b2)
    av3 = av.reshape(B, C, 1)
    bv3 = bv.reshape(B, C, 1)

    imp, y = pl.pallas_call(
        _imp_write_kernel,
        grid=(B, nl2),
        in_specs=[
            pl.BlockSpec((1, C, lb2), lambda b, l: (b, 0, l)),
            pl.BlockSpec((1, C, 1), lambda b, l: (b, 0, 0)),
            pl.BlockSpec((1, C, 1), lambda b, l: (b, 0, 0)),
        ],
        out_specs=[
            pl.BlockSpec((1, C, 1), lambda b, l: (b, 0, 0)),
            pl.BlockSpec((1, C, lb2), lambda b, l: (b, 0, l)),
        ],
        out_shape=[
            jax.ShapeDtypeStruct((B, C, 1), f32),
            jax.ShapeDtypeStruct((B, C, L), f32),
        ],
        compiler_params=pltpu.CompilerParams(
            dimension_semantics=("parallel", "arbitrary"),
            vmem_limit_bytes=50 * 1024 * 1024,
        ),
        name="duan_imp",
    )(x, av3, bv3)
    impl_t = jnp.transpose(imp, (0, 2, 1))

    y_out = pl.pallas_call(
        functools.partial(_mask_zero_kernel, k=k, n_drop=C - k, n_b=B),
        grid=(B,),
        in_specs=[
            pl.BlockSpec((1, C, 1), lambda b: (b, 0, 0)),
            pl.BlockSpec((1, 1, C), lambda b: (b, 0, 0)),
            pl.BlockSpec(memory_space=pl.ANY),
        ],
        out_specs=pl.BlockSpec(memory_space=pl.ANY),
        out_shape=jax.ShapeDtypeStruct((B, C, L), f32),
        input_output_aliases={2: 0},
        scratch_shapes=[
            pltpu.VMEM((1, C), jnp.int32),
            pltpu.SMEM((1, C), jnp.int32),
            pltpu.VMEM((L,), f32),
            pltpu.SemaphoreType.DMA,
            pltpu.SemaphoreType.DMA,
        ],
        compiler_params=pltpu.CompilerParams(
            dimension_semantics=("arbitrary",)),
        name="duan_mask_zero",
    )(imp, impl_t, y)
    return y_out.astype(x16.dtype)
